# baseline (speedup 1.0000x reference)
import jax
import jax.numpy as jnp
from jax import lax
from jax.experimental import pallas as pl
from jax.experimental.pallas import tpu as pltpu
from jax.experimental.pallas import tpu_sc as plsc

N_CORES = 2
N_SUBCORES = 16
L = 16

B = 16384
F = 100
M = 49
D = 32

FH = F // 2
MP = M * FH
RW = D // 2 + 1
BT = B // 128
BTW = BT // N_SUBCORES
DT = D // 8


def _splat(s):
    return lax.broadcast_in_dim(s, (L,), ())


def _body(x_hbm, tab_hbm, e0_hbm, hinv_hbm, out_hbm,
          tab_v, xb, e0_v, hinv_v, sb0, sb1, so0, so1):
    cid = lax.axis_index("c")
    sid = lax.axis_index("s")
    f0 = cid * FH
    bw0 = sid * (BTW * 128)
    bt0 = sid * BTW

    pltpu.sync_copy(tab_hbm.at[cid], tab_v)
    pltpu.sync_copy(e0_hbm, e0_v)
    pltpu.sync_copy(hinv_hbm, hinv_v)

    iot = lax.iota(jnp.int32, L)
    zero16 = jnp.zeros((L,), jnp.int32)

    def btloop(bt, _):
        pltpu.sync_copy(x_hbm.at[pl.ds(bw0 + bt * 128, 128)], xb)

        def floop2(fi, _):
            for j, (sb, so) in enumerate(((sb0, so0), (sb1, so1))):
                fl = fi * 2 + j
                fg = fl + f0
                fgv = _splat(fg)
                e0f = plsc.load_gather(e0_v, [fgv])
                hif = plsc.load_gather(hinv_v, [fgv])
                rowbase = _splat(fl * M)
                dst = out_hbm.at[cid, fl, :, pl.ds(bt0 + bt, 1)]

                @pl.when(jnp.logical_or(fl >= 2, bt > 0))
                def _():
                    pltpu.make_async_copy(sb, dst, so).wait()

                for g in range(128 // L):
                    xv = plsc.load_gather(xb, [iot + g * L, fgv])
                    v = (xv - e0f) * hif
                    bn = v.astype(jnp.int32)
                    bn = jnp.minimum(jnp.maximum(bn, 0), M - 2)
                    tv = jnp.clip(v - bn.astype(jnp.float32), 0.0, 1.0)
                    rl = bn + rowbase
                    rr = rl + 1

                    tp = plsc.pack(tv, tv, format=plsc.PackFormat.INTERLEAVED)

                    @plsc.parallel_loop(0, D // 2, unroll=8,
                                        carry=(zero16, jnp.int32(0),
                                               jnp.int32(0)))
                    def dloop(d, carry, rl=rl, rr=rr, tp=tp, sb=sb, g=g):
                        dv, dhi, dlo = carry
                        lv = plsc.bitcast(
                            plsc.load_gather(tab_v, [rl, dv]), jnp.bfloat16)
                        rv = plsc.bitcast(
                            plsc.load_gather(tab_v, [rr, dv]), jnp.bfloat16)
                        o = lv + tp * (rv - lv)
                        oa, ob = plsc.unpack(
                            o, format=plsc.PackFormat.INTERLEAVED)
                        sb[dhi, 0, dlo, pl.ds(g * L, L)] = oa
                        sb[dhi, 0, dlo + 1, pl.ds(g * L, L)] = ob
                        nlo = dlo + 2
                        wrap = nlo >= 8
                        nlo = jnp.where(wrap, 0, nlo)
                        nhi = jnp.where(wrap, dhi + 1, dhi)
                        return dv + 1, nhi, nlo

                pltpu.async_copy(sb, dst, so)
            return 0

        lax.fori_loop(0, FH // 2, floop2, 0)
        return 0

    lax.fori_loop(0, BTW, btloop, 0)

    last = out_hbm.at[cid, 0, :, pl.ds(bt0, 1)]
    pltpu.make_async_copy(sb0, last, so0).wait()
    pltpu.make_async_copy(sb1, last, so1).wait()


@jax.jit
def kernel(x, bin_edges, boundary_embeddings):
    e0 = bin_edges[:, 0]
    h = bin_edges[:, 1] - bin_edges[:, 0]
    hinv = jnp.where(jnp.abs(h) < 1e-8, 1.0, 1.0 / h)
    emb_bf = boundary_embeddings.astype(jnp.bfloat16).reshape(F * M, D // 2, 2)
    tab_i = jax.lax.bitcast_convert_type(emb_bf, jnp.int32)
    tabp = jnp.pad(tab_i, ((0, 0), (0, 1))).reshape(N_CORES, MP, RW)

    mesh = plsc.VectorSubcoreMesh(core_axis_name="c", subcore_axis_name="s")
    run = pl.kernel(
        _body,
        out_type=jax.ShapeDtypeStruct((N_CORES, FH, DT, BT, 8, 128),
                                      jnp.float32),
        mesh=mesh,
        compiler_params=pltpu.CompilerParams(
            use_tc_tiling_on_sc=False, needs_layout_passes=False),
        scratch_types=[
            pltpu.VMEM((MP, RW), jnp.int32),
            pltpu.VMEM((128, F), jnp.float32),
            pltpu.VMEM((F,), jnp.float32),
            pltpu.VMEM((F,), jnp.float32),
            pltpu.VMEM((DT, 1, 8, 128), jnp.float32),
            pltpu.VMEM((DT, 1, 8, 128), jnp.float32),
            pltpu.SemaphoreType.DMA,
            pltpu.SemaphoreType.DMA,
        ],
    )
    phys = run(x, tabp, e0, hinv)
    p6 = phys.reshape(F, DT, BT, 8, 128)
    return p6.transpose(2, 4, 0, 1, 3).reshape(B, F, D)


# speedup vs baseline: 1.0935x; 1.0935x over previous
import jax
import jax.numpy as jnp
from jax import lax
from jax.experimental import pallas as pl
from jax.experimental.pallas import tpu as pltpu
from jax.experimental.pallas import tpu_sc as plsc

N_CORES = 2
N_SUBCORES = 16
L = 16

B = 16384
F = 100
M = 49
D = 32

FH = F // 2
MP = M * FH
RW = D // 2 + 1
BT = B // 128
BTW = BT // N_SUBCORES
DT = D // 8


def _splat(s):
    return lax.broadcast_in_dim(s, (L,), ())


def _body(x_hbm, tab_hbm, e0_hbm, hinv_hbm, out_hbm,
          tab_v, xb, e0_v, hinv_v, sb0, sb1, so0, so1):
    cid = lax.axis_index("c")
    sid = lax.axis_index("s")
    f0 = cid * FH
    bw0 = sid * (BTW * 128)
    bt0 = sid * BTW

    pltpu.sync_copy(tab_hbm.at[cid], tab_v)
    pltpu.sync_copy(e0_hbm, e0_v)
    pltpu.sync_copy(hinv_hbm, hinv_v)

    iot = lax.iota(jnp.int32, L)
    zero16 = jnp.zeros((L,), jnp.int32)

    def btloop(bt, _):
        pltpu.sync_copy(x_hbm.at[pl.ds(bw0 + bt * 128, 128)], xb)

        def floop2(fi, _):
            for j, (sb, so) in enumerate(((sb0, so0), (sb1, so1))):
                fl = fi * 2 + j
                fg = fl + f0
                fgv = _splat(fg)
                e0f = plsc.load_gather(e0_v, [fgv])
                hif = plsc.load_gather(hinv_v, [fgv])
                rowbase = _splat(fl * M)
                dst = out_hbm.at[cid, fl, :, pl.ds(bt0 + bt, 1)]

                @pl.when(jnp.logical_or(fl >= 2, bt > 0))
                def _():
                    pltpu.make_async_copy(sb, dst, so).wait()

                for g in range(128 // L):
                    xv = plsc.load_gather(xb, [iot + g * L, fgv])
                    v = (xv - e0f) * hif
                    bn = v.astype(jnp.int32)
                    bn = jnp.minimum(jnp.maximum(bn, 0), M - 2)
                    tv = jnp.clip(v - bn.astype(jnp.float32), 0.0, 1.0)
                    rl = bn + rowbase
                    rr = rl + 1

                    tp = plsc.pack(tv, tv, format=plsc.PackFormat.INTERLEAVED)

                    @plsc.parallel_loop(0, D // 2, unroll=4,
                                        carry=(zero16, jnp.int32(0),
                                               jnp.int32(0)))
                    def dloop(d, carry, rl=rl, rr=rr, tp=tp, sb=sb, g=g):
                        dv, dhi, dlo = carry
                        lv = plsc.bitcast(
                            plsc.load_gather(tab_v, [rl, dv]), jnp.bfloat16)
                        rv = plsc.bitcast(
                            plsc.load_gather(tab_v, [rr, dv]), jnp.bfloat16)
                        o = lv + tp * (rv - lv)
                        oa, ob = plsc.unpack(
                            o, format=plsc.PackFormat.INTERLEAVED)
                        sb[dhi, 0, dlo, pl.ds(g * L, L)] = oa
                        sb[dhi, 0, dlo + 1, pl.ds(g * L, L)] = ob
                        nlo = dlo + 2
                        wrap = nlo >= 8
                        nlo = jnp.where(wrap, 0, nlo)
                        nhi = jnp.where(wrap, dhi + 1, dhi)
                        return dv + 1, nhi, nlo

                pltpu.async_copy(sb, dst, so)
            return 0

        lax.fori_loop(0, FH // 2, floop2, 0)
        return 0

    lax.fori_loop(0, BTW, btloop, 0)

    last = out_hbm.at[cid, 0, :, pl.ds(bt0, 1)]
    pltpu.make_async_copy(sb0, last, so0).wait()
    pltpu.make_async_copy(sb1, last, so1).wait()


@jax.jit
def kernel(x, bin_edges, boundary_embeddings):
    e0 = bin_edges[:, 0]
    h = bin_edges[:, 1] - bin_edges[:, 0]
    hinv = jnp.where(jnp.abs(h) < 1e-8, 1.0, 1.0 / h)
    emb_bf = boundary_embeddings.astype(jnp.bfloat16).reshape(F * M, D // 2, 2)
    tab_i = jax.lax.bitcast_convert_type(emb_bf, jnp.int32)
    tabp = jnp.pad(tab_i, ((0, 0), (0, 1))).reshape(N_CORES, MP, RW)

    mesh = plsc.VectorSubcoreMesh(core_axis_name="c", subcore_axis_name="s")
    run = pl.kernel(
        _body,
        out_type=jax.ShapeDtypeStruct((N_CORES, FH, DT, BT, 8, 128),
                                      jnp.float32),
        mesh=mesh,
        compiler_params=pltpu.CompilerParams(
            use_tc_tiling_on_sc=False, needs_layout_passes=False),
        scratch_types=[
            pltpu.VMEM((MP, RW), jnp.int32),
            pltpu.VMEM((128, F), jnp.float32),
            pltpu.VMEM((F,), jnp.float32),
            pltpu.VMEM((F,), jnp.float32),
            pltpu.VMEM((DT, 1, 8, 128), jnp.float32),
            pltpu.VMEM((DT, 1, 8, 128), jnp.float32),
            pltpu.SemaphoreType.DMA,
            pltpu.SemaphoreType.DMA,
        ],
    )
    phys = run(x, tabp, e0, hinv)
    p6 = phys.reshape(F, DT, BT, 8, 128)
    return p6.transpose(2, 4, 0, 1, 3).reshape(B, F, D)


# speedup vs baseline: 1.1066x; 1.0120x over previous
import jax
import jax.numpy as jnp
from jax import lax
from jax.experimental import pallas as pl
from jax.experimental.pallas import tpu as pltpu
from jax.experimental.pallas import tpu_sc as plsc

N_CORES = 2
N_SUBCORES = 16
L = 16

B = 16384
F = 100
M = 49
D = 32

FH = F // 2
MP = M * FH
RW = D // 2 + 1
BT = B // 128
BTW = BT // N_SUBCORES
DT = D // 8


def _splat(s):
    return lax.broadcast_in_dim(s, (L,), ())


def _body(x_hbm, tab_hbm, e0_hbm, hinv_hbm, out_hbm,
          tab_v, xba, xbb, e0_v, hinv_v, sb0, sb1, so0, so1, sxa, sxb):
    cid = lax.axis_index("c")
    sid = lax.axis_index("s")
    f0 = cid * FH
    bw0 = sid * (BTW * 128)
    bt0 = sid * BTW

    pltpu.sync_copy(tab_hbm.at[cid], tab_v)
    pltpu.sync_copy(e0_hbm, e0_v)
    pltpu.sync_copy(hinv_hbm, hinv_v)

    iot = lax.iota(jnp.int32, L)
    zero16 = jnp.zeros((L,), jnp.int32)

    pltpu.async_copy(x_hbm.at[pl.ds(bw0, 128)], xba, sxa)
    xbufs = ((xba, sxa, xbb, sxb), (xbb, sxb, xba, sxa))

    def btloop2(bi2, _):
      for jj, (xb, sx, nxb, nsx) in enumerate(xbufs):
        bt = bi2 * 2 + jj
        nbt = bt + 1
        nbt = jnp.where(nbt >= BTW, 0, nbt)
        pltpu.async_copy(x_hbm.at[pl.ds(bw0 + nbt * 128, 128)], nxb, nsx)
        pltpu.make_async_copy(
            x_hbm.at[pl.ds(bw0 + bt * 128, 128)], xb, sx).wait()

        def floop2(fi, _, xb=xb, bt=bt):
            for j, (sb, so) in enumerate(((sb0, so0), (sb1, so1))):
                fl = fi * 2 + j
                fg = fl + f0
                fgv = _splat(fg)
                e0f = plsc.load_gather(e0_v, [fgv])
                hif = plsc.load_gather(hinv_v, [fgv])
                rowbase = _splat(fl * M)
                dst = out_hbm.at[cid, fl, :, pl.ds(bt0 + bt, 1)]

                @pl.when(jnp.logical_or(fl >= 2, bt > 0))
                def _():
                    pltpu.make_async_copy(sb, dst, so).wait()

                for g in range(128 // L):
                    xv = plsc.load_gather(xb, [iot + g * L, fgv])
                    v = (xv - e0f) * hif
                    bn = v.astype(jnp.int32)
                    bn = jnp.minimum(jnp.maximum(bn, 0), M - 2)
                    tv = jnp.clip(v - bn.astype(jnp.float32), 0.0, 1.0)
                    rl = bn + rowbase
                    rr = rl + 1

                    tp = plsc.pack(tv, tv, format=plsc.PackFormat.INTERLEAVED)

                    @plsc.parallel_loop(0, D // 2, unroll=4,
                                        carry=(zero16, jnp.int32(0),
                                               jnp.int32(0)))
                    def dloop(d, carry, rl=rl, rr=rr, tp=tp, sb=sb, g=g):
                        dv, dhi, dlo = carry
                        lv = plsc.bitcast(
                            plsc.load_gather(tab_v, [rl, dv]), jnp.bfloat16)
                        rv = plsc.bitcast(
                            plsc.load_gather(tab_v, [rr, dv]), jnp.bfloat16)
                        o = lv + tp * (rv - lv)
                        oa, ob = plsc.unpack(
                            o, format=plsc.PackFormat.INTERLEAVED)
                        sb[dhi, 0, dlo, pl.ds(g * L, L)] = oa
                        sb[dhi, 0, dlo + 1, pl.ds(g * L, L)] = ob
                        nlo = dlo + 2
                        wrap = nlo >= 8
                        nlo = jnp.where(wrap, 0, nlo)
                        nhi = jnp.where(wrap, dhi + 1, dhi)
                        return dv + 1, nhi, nlo

                pltpu.async_copy(sb, dst, so)
            return 0

        lax.fori_loop(0, FH // 2, floop2, 0)
      return 0

    lax.fori_loop(0, BTW // 2, btloop2, 0)

    pltpu.make_async_copy(x_hbm.at[pl.ds(bw0, 128)], xba, sxa).wait()
    last = out_hbm.at[cid, 0, :, pl.ds(bt0, 1)]
    pltpu.make_async_copy(sb0, last, so0).wait()
    pltpu.make_async_copy(sb1, last, so1).wait()


@jax.jit
def kernel(x, bin_edges, boundary_embeddings):
    e0 = bin_edges[:, 0]
    h = bin_edges[:, 1] - bin_edges[:, 0]
    hinv = jnp.where(jnp.abs(h) < 1e-8, 1.0, 1.0 / h)
    emb_bf = boundary_embeddings.astype(jnp.bfloat16).reshape(F * M, D // 2, 2)
    tab_i = jax.lax.bitcast_convert_type(emb_bf, jnp.int32)
    tabp = jnp.pad(tab_i, ((0, 0), (0, 1))).reshape(N_CORES, MP, RW)

    mesh = plsc.VectorSubcoreMesh(core_axis_name="c", subcore_axis_name="s")
    run = pl.kernel(
        _body,
        out_type=jax.ShapeDtypeStruct((N_CORES, FH, DT, BT, 8, 128),
                                      jnp.float32),
        mesh=mesh,
        compiler_params=pltpu.CompilerParams(
            use_tc_tiling_on_sc=False, needs_layout_passes=False),
        scratch_types=[
            pltpu.VMEM((MP, RW), jnp.int32),
            pltpu.VMEM((128, F), jnp.float32),
            pltpu.VMEM((128, F), jnp.float32),
            pltpu.VMEM((F,), jnp.float32),
            pltpu.VMEM((F,), jnp.float32),
            pltpu.VMEM((DT, 1, 8, 128), jnp.float32),
            pltpu.VMEM((DT, 1, 8, 128), jnp.float32),
            pltpu.SemaphoreType.DMA,
            pltpu.SemaphoreType.DMA,
            pltpu.SemaphoreType.DMA,
            pltpu.SemaphoreType.DMA,
        ],
    )
    phys = run(x, tabp, e0, hinv)
    p6 = phys.reshape(F, DT, BT, 8, 128)
    return p6.transpose(2, 4, 0, 1, 3).reshape(B, F, D)
